# baseline (speedup 1.0000x reference)
import functools

import jax
import jax.numpy as jnp
from jax import lax
from jax.experimental import pallas as pl
from jax.experimental.pallas import tpu as pltpu

NEG_SLOPE = 0.2
BN_EPS = 1e-5
MASK_VAL = -1e30
CHUNK = 1024
UNROLL = 8


def _round_up(v, m):
    return (v + m - 1) // m * m


def _pad2(a, rows, cols):
    return jnp.pad(a, ((0, rows - a.shape[0]), (0, cols - a.shape[1])))


def _project_kernel(x_ref, w_ref, emb_ref, att_i_ref, att_em_i_ref,
                    att_j_ref, att_em_j_ref, xw_ref, a_ref, b_ref):
    xw = jnp.dot(x_ref[...].astype(jnp.bfloat16), w_ref[...],
                 preferred_element_type=jnp.float32)
    emb = emb_ref[...]
    a = (jnp.sum(xw * att_i_ref[...], axis=1, keepdims=True)
         + jnp.sum(emb * att_em_i_ref[...], axis=1, keepdims=True))
    b = (jnp.sum(xw * att_j_ref[...], axis=1, keepdims=True)
         + jnp.sum(emb * att_em_j_ref[...], axis=1, keepdims=True))
    xw_ref[...] = xw.astype(jnp.bfloat16)
    a_ref[...] = a
    b_ref[...] = b


def _attend_kernel(starts_ref, keys_ref, a_ref, b_ref, xw_ref, bias_ref,
                   rmask_ref, out_ref, psum_ref, psumsq_ref,
                   m0, m1, m2, m3, kbuf, sems, *, tm, n_pad, shift):
    bufs = (m0, m1, m2, m3)
    i = pl.program_id(0)
    row_base = i * tm
    start = starts_ref[i]
    end = starts_ref[i + 1]
    base = (start >> 7) << 7
    span = end - base
    nch = (span + CHUNK - 1) // CHUNK

    def _copy(c, slot):
        off = pl.multiple_of(base + c * CHUNK, 128)
        return pltpu.make_async_copy(
            keys_ref.at[pl.ds(off, CHUNK)],
            kbuf.at[slot], sems.at[slot])

    @pl.when(nch > 0)
    def _():
        _copy(0, 0).start()

    for b in bufs:
        b[...] = jnp.zeros_like(b)

    siota = lax.broadcasted_iota(jnp.int32, (32, 128), 0)
    liota = lax.broadcasted_iota(jnp.int32, (32, 128), 1)

    def chunk_body(c, _):
        slot = c & 1

        @pl.when(c + 1 < nch)
        def _():
            _copy(c + 1, 1 - slot).start()

        pltpu.make_async_copy(kbuf.at[slot], kbuf.at[slot],
                              sems.at[slot]).wait()

        def edge_body(t, _):
            e0 = t * UNROLL
            for u in range(UNROLL):
                k = kbuf[slot, e0 + u]
                r = (k >> shift) - row_base
                col = k & (n_pad - 1)
                bad = jnp.logical_or(r < 0, r >= tm)
                rr = jnp.where(bad, tm, r)
                rb = pl.multiple_of((rr >> 5) << 5, 32)
                g = pl.multiple_of((col >> 7) << 7, 128)
                buf = bufs[u & 3]
                hot = jnp.logical_and(siota == (rr & 31), liota == (col & 127))
                tile = buf[pl.ds(rb, 32), pl.ds(g, 128)]
                buf[pl.ds(rb, 32), pl.ds(g, 128)] = tile | hot.astype(jnp.int8)
            return 0

        lax.fori_loop(0, CHUNK // UNROLL, edge_body, 0)
        return 0

    lax.fori_loop(0, nch, chunk_body, 0)

    mask = ((m0[0:tm, :] | m1[0:tm, :]) | (m2[0:tm, :] | m3[0:tm, :])) != 0
    alpha = a_ref[...] + b_ref[...]
    alpha = jnp.maximum(alpha, NEG_SLOPE * alpha)
    masked = jnp.where(mask, alpha, MASK_VAL)
    m = jnp.max(masked, axis=1, keepdims=True)
    e = jnp.exp(masked - m)
    l = jnp.sum(e, axis=1, keepdims=True)
    acc = jnp.dot(e.astype(jnp.bfloat16), xw_ref[...],
                  preferred_element_type=jnp.float32)
    out = acc / l + bias_ref[...]
    out_ref[...] = out.astype(out_ref.dtype)
    m_out = out * rmask_ref[...]
    psum_ref[...] = jnp.sum(m_out, axis=0, keepdims=True)[None]
    psumsq_ref[...] = jnp.sum(m_out * out, axis=0, keepdims=True)[None]


def _bn_relu_kernel(out_ref, scale_ref, shift_ref, y_ref):
    y_ref[...] = jnp.maximum(
        out_ref[...].astype(jnp.float32) * scale_ref[...] + shift_ref[...], 0.0)


def kernel(x, edge_index, embedding, w, att_i, att_j, att_em_i, att_em_j,
           bias, gamma, beta):
    n, cin = x.shape
    cout = w.shape[1]

    cin_p = _round_up(cin, 128)
    cout_p = _round_up(cout, 128)
    n_pad = _round_up(n, 512)
    if n_pad & (n_pad - 1):
        n_pad = 1 << (n_pad - 1).bit_length()
    shift = n_pad.bit_length() - 1

    tm = 256
    while n_pad % tm:
        tm //= 2
    nt = n_pad // tm

    tmp = 512
    while n_pad % tmp:
        tmp //= 2
    ntp = n_pad // tmp

    x_p = _pad2(x, n_pad, cin_p)
    emb_p = _pad2(embedding, n_pad, cout_p)
    w_p = _pad2(w, cin_p, cout_p).astype(jnp.bfloat16)
    att_i_p = _pad2(att_i, 1, cout_p)
    att_j_p = _pad2(att_j, 1, cout_p)
    att_em_i_p = _pad2(att_em_i, 1, cout_p)
    att_em_j_p = _pad2(att_em_j, 1, cout_p)
    bias_p = _pad2(bias, 1, cout_p)
    gamma_p = _pad2(gamma, 1, cout_p)
    beta_p = _pad2(beta, 1, cout_p)

    src_e, dst_e = edge_index[0], edge_index[1]
    diag = jnp.arange(n_pad, dtype=jnp.int32) * (n_pad + 1)
    keys = jnp.sort(jnp.concatenate([dst_e * n_pad + src_e, diag]))
    e2 = keys.shape[0]
    sent = n_pad * n_pad
    pad_len = _round_up(e2 + CHUNK + 128, 128) - e2
    keys_p = jnp.concatenate([keys, jnp.full((pad_len,), sent, jnp.int32)])
    bounds = (jnp.arange(nt + 1) * (tm * n_pad)).astype(jnp.int32)
    starts = jnp.searchsorted(keys, bounds).astype(jnp.int32)

    rowmask = (jnp.arange(n_pad) < n).astype(jnp.float32).reshape(n_pad, 1)

    vmem_lim = 48 * 1024 * 1024
    cp_par = pltpu.CompilerParams(dimension_semantics=("parallel",),
                                  vmem_limit_bytes=vmem_lim)

    xw, a_col, b_col = pl.pallas_call(
        _project_kernel,
        out_shape=(jax.ShapeDtypeStruct((n_pad, cout_p), jnp.bfloat16),
                   jax.ShapeDtypeStruct((n_pad, 1), jnp.float32),
                   jax.ShapeDtypeStruct((n_pad, 1), jnp.float32)),
        grid=(ntp,),
        in_specs=[pl.BlockSpec((tmp, cin_p), lambda i: (i, 0)),
                  pl.BlockSpec((cin_p, cout_p), lambda i: (0, 0)),
                  pl.BlockSpec((tmp, cout_p), lambda i: (i, 0)),
                  pl.BlockSpec((1, cout_p), lambda i: (0, 0)),
                  pl.BlockSpec((1, cout_p), lambda i: (0, 0)),
                  pl.BlockSpec((1, cout_p), lambda i: (0, 0)),
                  pl.BlockSpec((1, cout_p), lambda i: (0, 0))],
        out_specs=(pl.BlockSpec((tmp, cout_p), lambda i: (i, 0)),
                   pl.BlockSpec((tmp, 1), lambda i: (i, 0)),
                   pl.BlockSpec((tmp, 1), lambda i: (i, 0))),
        compiler_params=cp_par,
    )(x_p, w_p, emb_p, att_i_p, att_em_i_p, att_j_p, att_em_j_p)

    b_row = b_col.reshape(1, n_pad)

    out_pre, psum, psumsq = pl.pallas_call(
        functools.partial(_attend_kernel, tm=tm, n_pad=n_pad, shift=shift),
        out_shape=(jax.ShapeDtypeStruct((n_pad, cout_p), jnp.bfloat16),
                   jax.ShapeDtypeStruct((nt, 1, cout_p), jnp.float32),
                   jax.ShapeDtypeStruct((nt, 1, cout_p), jnp.float32)),
        grid_spec=pltpu.PrefetchScalarGridSpec(
            num_scalar_prefetch=1,
            grid=(nt,),
            in_specs=[pl.BlockSpec(memory_space=pl.ANY),
                      pl.BlockSpec((tm, 1), lambda i, s: (i, 0)),
                      pl.BlockSpec((1, n_pad), lambda i, s: (0, 0)),
                      pl.BlockSpec((n_pad, cout_p), lambda i, s: (0, 0)),
                      pl.BlockSpec((1, cout_p), lambda i, s: (0, 0)),
                      pl.BlockSpec((tm, 1), lambda i, s: (i, 0))],
            out_specs=(pl.BlockSpec((tm, cout_p), lambda i, s: (i, 0)),
                       pl.BlockSpec((1, 1, cout_p), lambda i, s: (i, 0, 0)),
                       pl.BlockSpec((1, 1, cout_p), lambda i, s: (i, 0, 0))),
            scratch_shapes=[pltpu.VMEM((tm + 32, n_pad), jnp.int8),
                            pltpu.VMEM((tm + 32, n_pad), jnp.int8),
                            pltpu.VMEM((tm + 32, n_pad), jnp.int8),
                            pltpu.VMEM((tm + 32, n_pad), jnp.int8),
                            pltpu.SMEM((2, CHUNK), jnp.int32),
                            pltpu.SemaphoreType.DMA((2,))],
        ),
        compiler_params=cp_par,
    )(starts, keys_p, a_col, b_row, xw, bias_p, rowmask)

    s = jnp.sum(psum, axis=(0, 1))
    ssq = jnp.sum(psumsq, axis=(0, 1))
    mean = s / n
    var = jnp.maximum(ssq / n - mean * mean, 0.0)
    inv = lax.rsqrt(var + BN_EPS)
    scale = (gamma_p[0] * inv).reshape(1, cout_p)
    shift_v = (beta_p[0] - mean * gamma_p[0] * inv).reshape(1, cout_p)

    y = pl.pallas_call(
        _bn_relu_kernel,
        out_shape=jax.ShapeDtypeStruct((n_pad, cout_p), jnp.float32),
        grid=(ntp,),
        in_specs=[pl.BlockSpec((tmp, cout_p), lambda i: (i, 0)),
                  pl.BlockSpec((1, cout_p), lambda i: (0, 0)),
                  pl.BlockSpec((1, cout_p), lambda i: (0, 0))],
        out_specs=pl.BlockSpec((tmp, cout_p), lambda i: (i, 0)),
        compiler_params=cp_par,
    )(out_pre, scale, shift_v)

    return y[:n, :cout]


# speedup vs baseline: 2.1484x; 2.1484x over previous
import jax
import jax.numpy as jnp
from jax import lax
from jax.experimental import pallas as pl
from jax.experimental.pallas import tpu as pltpu

NEG_SLOPE = 0.2
BN_EPS = 1e-5
MASK_VAL = -1e30


def _round_up(v, m):
    return (v + m - 1) // m * m


def _pad2(a, rows, cols):
    return jnp.pad(a, ((0, rows - a.shape[0]), (0, cols - a.shape[1])))


def _project_kernel(x_ref, w_ref, emb_ref, att_i_ref, att_em_i_ref,
                    att_j_ref, att_em_j_ref, xw_ref, a_ref, b_ref):
    xw = jnp.dot(x_ref[...].astype(jnp.bfloat16), w_ref[...],
                 preferred_element_type=jnp.float32)
    emb = emb_ref[...]
    a = (jnp.sum(xw * att_i_ref[...], axis=1, keepdims=True)
         + jnp.sum(emb * att_em_i_ref[...], axis=1, keepdims=True))
    b = (jnp.sum(xw * att_j_ref[...], axis=1, keepdims=True)
         + jnp.sum(emb * att_em_j_ref[...], axis=1, keepdims=True))
    xw_ref[...] = xw.astype(jnp.bfloat16)
    a_ref[...] = a
    b_ref[...] = b


def _attend_kernel(adj_ref, a_ref, b_ref, xw_ref, bias_ref, rmask_ref,
                   out_ref, psum_ref, psumsq_ref):
    mask = adj_ref[...] != 0
    alpha = a_ref[...] + b_ref[...]
    alpha = jnp.maximum(alpha, NEG_SLOPE * alpha)
    masked = jnp.where(mask, alpha, MASK_VAL)
    m = jnp.max(masked, axis=1, keepdims=True)
    e = jnp.exp(masked - m)
    l = jnp.sum(e, axis=1, keepdims=True)
    acc = jnp.dot(e.astype(jnp.bfloat16), xw_ref[...],
                  preferred_element_type=jnp.float32)
    out = acc / l + bias_ref[...]
    out_ref[...] = out.astype(out_ref.dtype)
    m_out = out * rmask_ref[...]
    psum_ref[...] = jnp.sum(m_out, axis=0, keepdims=True)[None]
    psumsq_ref[...] = jnp.sum(m_out * out, axis=0, keepdims=True)[None]


def _bn_relu_kernel(out_ref, scale_ref, shift_ref, y_ref):
    y_ref[...] = jnp.maximum(
        out_ref[...].astype(jnp.float32) * scale_ref[...] + shift_ref[...], 0.0)


def kernel(x, edge_index, embedding, w, att_i, att_j, att_em_i, att_em_j,
           bias, gamma, beta):
    n, cin = x.shape
    cout = w.shape[1]

    cin_p = _round_up(cin, 128)
    cout_p = _round_up(cout, 128)
    n_pad = _round_up(n, 512)

    tm = 256
    while n_pad % tm:
        tm //= 2
    nt = n_pad // tm

    tmp = 512
    while n_pad % tmp:
        tmp //= 2
    ntp = n_pad // tmp

    x_p = _pad2(x, n_pad, cin_p)
    emb_p = _pad2(embedding, n_pad, cout_p)
    w_p = _pad2(w, cin_p, cout_p).astype(jnp.bfloat16)
    att_i_p = _pad2(att_i, 1, cout_p)
    att_j_p = _pad2(att_j, 1, cout_p)
    att_em_i_p = _pad2(att_em_i, 1, cout_p)
    att_em_j_p = _pad2(att_em_j, 1, cout_p)
    bias_p = _pad2(bias, 1, cout_p)
    gamma_p = _pad2(gamma, 1, cout_p)
    beta_p = _pad2(beta, 1, cout_p)

    src_e, dst_e = edge_index[0], edge_index[1]
    diag = jnp.arange(n_pad, dtype=jnp.int32) * (n_pad + 1)
    keys = jnp.sort(jnp.concatenate([dst_e * n_pad + src_e, diag]))
    sent = jnp.int32(n_pad * n_pad)
    dup = jnp.concatenate([jnp.zeros((1,), jnp.bool_), keys[1:] == keys[:-1]])
    oob = sent + jnp.arange(keys.shape[0], dtype=jnp.int32)
    keys = jnp.where(dup, oob, keys)
    adj = (jnp.zeros((n_pad * n_pad,), jnp.float32)
           .at[keys].add(1.0, indices_are_sorted=True, unique_indices=True,
                         mode="drop")
           .reshape(n_pad, n_pad))

    rowmask = (jnp.arange(n_pad) < n).astype(jnp.float32).reshape(n_pad, 1)

    vmem_lim = 48 * 1024 * 1024
    cp_par = pltpu.CompilerParams(dimension_semantics=("parallel",),
                                  vmem_limit_bytes=vmem_lim)

    xw, a_col, b_col = pl.pallas_call(
        _project_kernel,
        out_shape=(jax.ShapeDtypeStruct((n_pad, cout_p), jnp.bfloat16),
                   jax.ShapeDtypeStruct((n_pad, 1), jnp.float32),
                   jax.ShapeDtypeStruct((n_pad, 1), jnp.float32)),
        grid=(ntp,),
        in_specs=[pl.BlockSpec((tmp, cin_p), lambda i: (i, 0)),
                  pl.BlockSpec((cin_p, cout_p), lambda i: (0, 0)),
                  pl.BlockSpec((tmp, cout_p), lambda i: (i, 0)),
                  pl.BlockSpec((1, cout_p), lambda i: (0, 0)),
                  pl.BlockSpec((1, cout_p), lambda i: (0, 0)),
                  pl.BlockSpec((1, cout_p), lambda i: (0, 0)),
                  pl.BlockSpec((1, cout_p), lambda i: (0, 0))],
        out_specs=(pl.BlockSpec((tmp, cout_p), lambda i: (i, 0)),
                   pl.BlockSpec((tmp, 1), lambda i: (i, 0)),
                   pl.BlockSpec((tmp, 1), lambda i: (i, 0))),
        compiler_params=cp_par,
    )(x_p, w_p, emb_p, att_i_p, att_em_i_p, att_j_p, att_em_j_p)

    b_row = b_col.reshape(1, n_pad)

    out_pre, psum, psumsq = pl.pallas_call(
        _attend_kernel,
        out_shape=(jax.ShapeDtypeStruct((n_pad, cout_p), jnp.bfloat16),
                   jax.ShapeDtypeStruct((nt, 1, cout_p), jnp.float32),
                   jax.ShapeDtypeStruct((nt, 1, cout_p), jnp.float32)),
        grid=(nt,),
        in_specs=[pl.BlockSpec((tm, n_pad), lambda i: (i, 0)),
                  pl.BlockSpec((tm, 1), lambda i: (i, 0)),
                  pl.BlockSpec((1, n_pad), lambda i: (0, 0)),
                  pl.BlockSpec((n_pad, cout_p), lambda i: (0, 0)),
                  pl.BlockSpec((1, cout_p), lambda i: (0, 0)),
                  pl.BlockSpec((tm, 1), lambda i: (i, 0))],
        out_specs=(pl.BlockSpec((tm, cout_p), lambda i: (i, 0)),
                   pl.BlockSpec((1, 1, cout_p), lambda i: (i, 0, 0)),
                   pl.BlockSpec((1, 1, cout_p), lambda i: (i, 0, 0))),
        compiler_params=cp_par,
    )(adj, a_col, b_row, xw, bias_p, rowmask)

    s = jnp.sum(psum, axis=(0, 1))
    ssq = jnp.sum(psumsq, axis=(0, 1))
    mean = s / n
    var = jnp.maximum(ssq / n - mean * mean, 0.0)
    inv = lax.rsqrt(var + BN_EPS)
    scale = (gamma_p[0] * inv).reshape(1, cout_p)
    shift = (beta_p[0] - mean * gamma_p[0] * inv).reshape(1, cout_p)

    y = pl.pallas_call(
        _bn_relu_kernel,
        out_shape=jax.ShapeDtypeStruct((n_pad, cout_p), jnp.float32),
        grid=(ntp,),
        in_specs=[pl.BlockSpec((tmp, cout_p), lambda i: (i, 0)),
                  pl.BlockSpec((1, cout_p), lambda i: (0, 0)),
                  pl.BlockSpec((1, cout_p), lambda i: (0, 0))],
        out_specs=pl.BlockSpec((tmp, cout_p), lambda i: (i, 0)),
        compiler_params=cp_par,
    )(out_pre, scale, shift)

    return y[:n, :cout]


# speedup vs baseline: 2.1569x; 1.0039x over previous
import jax
import jax.numpy as jnp
from jax import lax
from jax.experimental import pallas as pl
from jax.experimental.pallas import tpu as pltpu

NEG_SLOPE = 0.2
BN_EPS = 1e-5
MASK_VAL = -1e30


def _round_up(v, m):
    return (v + m - 1) // m * m


def _pad2(a, rows, cols):
    return jnp.pad(a, ((0, rows - a.shape[0]), (0, cols - a.shape[1])))


def _project_kernel(x_ref, w_ref, emb_ref, att_i_ref, att_em_i_ref,
                    att_j_ref, att_em_j_ref, xw_ref, a_ref, b_ref):
    xw = jnp.dot(x_ref[...].astype(jnp.bfloat16), w_ref[...],
                 preferred_element_type=jnp.float32)
    emb = emb_ref[...]
    a = (jnp.sum(xw * att_i_ref[...], axis=1, keepdims=True)
         + jnp.sum(emb * att_em_i_ref[...], axis=1, keepdims=True))
    b = (jnp.sum(xw * att_j_ref[...], axis=1, keepdims=True)
         + jnp.sum(emb * att_em_j_ref[...], axis=1, keepdims=True))
    xw_ref[...] = xw.astype(jnp.bfloat16)
    a_ref[...] = a
    b_ref[...] = b


def _attend_kernel(adj_ref, a_ref, b_ref, xw_ref, bias_ref, rmask_ref,
                   out_ref, psum_ref, psumsq_ref):
    mask = adj_ref[...] != 0
    alpha = a_ref[...] + b_ref[...]
    alpha = jnp.maximum(alpha, NEG_SLOPE * alpha)
    masked = jnp.where(mask, alpha, MASK_VAL)
    m = jnp.max(masked, axis=1, keepdims=True)
    e = jnp.exp(masked - m)
    l = jnp.sum(e, axis=1, keepdims=True)
    acc = jnp.dot(e.astype(jnp.bfloat16), xw_ref[...],
                  preferred_element_type=jnp.float32)
    out = acc / l + bias_ref[...]
    out_ref[...] = out.astype(out_ref.dtype)
    m_out = out * rmask_ref[...]
    psum_ref[...] = jnp.sum(m_out, axis=0, keepdims=True)[None]
    psumsq_ref[...] = jnp.sum(m_out * out, axis=0, keepdims=True)[None]


def _bn_relu_kernel(out_ref, scale_ref, shift_ref, y_ref):
    y_ref[...] = jnp.maximum(
        out_ref[...].astype(jnp.float32) * scale_ref[...] + shift_ref[...], 0.0)


def kernel(x, edge_index, embedding, w, att_i, att_j, att_em_i, att_em_j,
           bias, gamma, beta):
    n, cin = x.shape
    cout = w.shape[1]

    cin_p = _round_up(cin, 128)
    cout_p = _round_up(cout, 128)
    n_pad = _round_up(n, 512)

    tm = 256
    while n_pad % tm:
        tm //= 2
    nt = n_pad // tm

    tmp = 512
    while n_pad % tmp:
        tmp //= 2
    ntp = n_pad // tmp

    x_p = _pad2(x, n_pad, cin_p)
    emb_p = _pad2(embedding, n_pad, cout_p)
    w_p = _pad2(w, cin_p, cout_p).astype(jnp.bfloat16)
    att_i_p = _pad2(att_i, 1, cout_p)
    att_j_p = _pad2(att_j, 1, cout_p)
    att_em_i_p = _pad2(att_em_i, 1, cout_p)
    att_em_j_p = _pad2(att_em_j, 1, cout_p)
    bias_p = _pad2(bias, 1, cout_p)
    gamma_p = _pad2(gamma, 1, cout_p)
    beta_p = _pad2(beta, 1, cout_p)

    src_e, dst_e = edge_index[0], edge_index[1]
    diag = jnp.arange(n_pad, dtype=jnp.int32) * (n_pad + 1)
    keys = jnp.concatenate([dst_e * n_pad + src_e, diag])
    adj = (jnp.zeros((n_pad * n_pad,), jnp.float32).at[keys].add(1.0)
           .reshape(n_pad, n_pad))

    rowmask = (jnp.arange(n_pad) < n).astype(jnp.float32).reshape(n_pad, 1)

    vmem_lim = 48 * 1024 * 1024
    cp_par = pltpu.CompilerParams(dimension_semantics=("parallel",),
                                  vmem_limit_bytes=vmem_lim)

    xw, a_col, b_col = pl.pallas_call(
        _project_kernel,
        out_shape=(jax.ShapeDtypeStruct((n_pad, cout_p), jnp.bfloat16),
                   jax.ShapeDtypeStruct((n_pad, 1), jnp.float32),
                   jax.ShapeDtypeStruct((n_pad, 1), jnp.float32)),
        grid=(ntp,),
        in_specs=[pl.BlockSpec((tmp, cin_p), lambda i: (i, 0)),
                  pl.BlockSpec((cin_p, cout_p), lambda i: (0, 0)),
                  pl.BlockSpec((tmp, cout_p), lambda i: (i, 0)),
                  pl.BlockSpec((1, cout_p), lambda i: (0, 0)),
                  pl.BlockSpec((1, cout_p), lambda i: (0, 0)),
                  pl.BlockSpec((1, cout_p), lambda i: (0, 0)),
                  pl.BlockSpec((1, cout_p), lambda i: (0, 0))],
        out_specs=(pl.BlockSpec((tmp, cout_p), lambda i: (i, 0)),
                   pl.BlockSpec((tmp, 1), lambda i: (i, 0)),
                   pl.BlockSpec((tmp, 1), lambda i: (i, 0))),
        compiler_params=cp_par,
    )(x_p, w_p, emb_p, att_i_p, att_em_i_p, att_j_p, att_em_j_p)

    b_row = b_col.reshape(1, n_pad)

    out_pre, psum, psumsq = pl.pallas_call(
        _attend_kernel,
        out_shape=(jax.ShapeDtypeStruct((n_pad, cout_p), jnp.bfloat16),
                   jax.ShapeDtypeStruct((nt, 1, cout_p), jnp.float32),
                   jax.ShapeDtypeStruct((nt, 1, cout_p), jnp.float32)),
        grid=(nt,),
        in_specs=[pl.BlockSpec((tm, n_pad), lambda i: (i, 0)),
                  pl.BlockSpec((tm, 1), lambda i: (i, 0)),
                  pl.BlockSpec((1, n_pad), lambda i: (0, 0)),
                  pl.BlockSpec((n_pad, cout_p), lambda i: (0, 0)),
                  pl.BlockSpec((1, cout_p), lambda i: (0, 0)),
                  pl.BlockSpec((tm, 1), lambda i: (i, 0))],
        out_specs=(pl.BlockSpec((tm, cout_p), lambda i: (i, 0)),
                   pl.BlockSpec((1, 1, cout_p), lambda i: (i, 0, 0)),
                   pl.BlockSpec((1, 1, cout_p), lambda i: (i, 0, 0))),
        compiler_params=cp_par,
    )(adj, a_col, b_row, xw, bias_p, rowmask)

    s = jnp.sum(psum, axis=(0, 1))
    ssq = jnp.sum(psumsq, axis=(0, 1))
    mean = s / n
    var = jnp.maximum(ssq / n - mean * mean, 0.0)
    inv = lax.rsqrt(var + BN_EPS)
    scale = (gamma_p[0] * inv).reshape(1, cout_p)
    shift = (beta_p[0] - mean * gamma_p[0] * inv).reshape(1, cout_p)

    y = pl.pallas_call(
        _bn_relu_kernel,
        out_shape=jax.ShapeDtypeStruct((n_pad, cout_p), jnp.float32),
        grid=(ntp,),
        in_specs=[pl.BlockSpec((tmp, cout_p), lambda i: (i, 0)),
                  pl.BlockSpec((1, cout_p), lambda i: (0, 0)),
                  pl.BlockSpec((1, cout_p), lambda i: (0, 0))],
        out_specs=pl.BlockSpec((tmp, cout_p), lambda i: (i, 0)),
        compiler_params=cp_par,
    )(out_pre, scale, shift)

    return y[:n, :cout]
